# Initial kernel scaffold; baseline (speedup 1.0000x reference)
import jax
import jax.numpy as jnp
from jax.experimental import pallas as pl


def kernel(edge_index1, edge_index2, in_feat, edge_feats, edge_feats1, W_ef1, b_ef1, W_ef2, b_ef2, b_conv1, b_conv2):
    raise NotImplementedError("write your pallas kernel here")


# speedup vs baseline: 1.0611x; 1.0611x over previous
import functools

import jax
import jax.numpy as jnp
from jax import lax
from jax.experimental import pallas as pl
from jax.experimental.pallas import tpu as pltpu
from jax.experimental.pallas import tpu_sc as plsc

N = 10000
E = 160000
IN = 128
H = 16
EF = 16

NC = 2
NS = 16
NW = NC * NS
E_PAD = 163840
EPW = E_PAD // NW
CH = 128
NCHUNK = EPW // CH
NPAD = 10240
NQ = 4
QR = NPAD // NQ
QG = QR + 8

_mesh = plsc.VectorSubcoreMesh(core_axis_name="c", subcore_axis_name="s")


def _make_gather(d):
    @functools.partial(
        pl.kernel,
        out_type=jax.ShapeDtypeStruct((E_PAD, d), jnp.float32),
        mesh=_mesh,
        scratch_types=[
            pltpu.VMEM((CH,), jnp.int32),
            pltpu.VMEM((CH, d), jnp.float32),
            pltpu.SemaphoreType.DMA,
        ],
    )
    def gather_k(tab_hbm, src_hbm, out_hbm, idx_v, rows_v, sem):
        wid = lax.axis_index("s") * NC + lax.axis_index("c")
        base = wid * EPW

        def body(j, carry):
            off = base + j * CH
            pltpu.sync_copy(src_hbm.at[pl.ds(off, CH)], idx_v)
            pltpu.async_copy(tab_hbm.at[idx_v], rows_v, sem).wait()
            pltpu.sync_copy(rows_v, out_hbm.at[pl.ds(off, CH)])
            return carry

        lax.fori_loop(0, NCHUNK, body, 0)

    return gather_k


_gather_x = _make_gather(IN)


@functools.partial(
    pl.kernel,
    out_type=jax.ShapeDtypeStruct((NW * NPAD * 32,), jnp.float32),
    mesh=_mesh,
    scratch_types=[
        pltpu.VMEM((CH,), jnp.int32),
        pltpu.VMEM((CH * 32,), jnp.float32),
        pltpu.VMEM((QG * 32,), jnp.float32),
    ],
)
def _scatter_k(r_hbm, dst_hbm, out_hbm, idx_v, rows_v, tab_v):
    c = lax.axis_index("c")
    s = lax.axis_index("s")
    w = s * NC + c
    base = w * EPW

    def qpass(q, carry):
        qlo = q * QR

        def zrow(i, carry2):
            tab_v[pl.ds(i * 16, 16)] = jnp.zeros((16,), jnp.float32)
            return carry2

        lax.fori_loop(0, QG * 2, zrow, 0)

        def body(j, carry2):
            off = base + j * CH
            pltpu.sync_copy(dst_hbm.at[pl.ds(off, CH)], idx_v)
            pltpu.sync_copy(r_hbm.at[pl.ds(off * 32, CH * 32)], rows_v)

            def grp16(g, carry3):
                dvec = idx_v[pl.ds(g * 16, 16)] - qlo
                inrange = jnp.logical_and(dvec >= 0, dvec < QR)
                dvec = jnp.where(inrange, dvec, QR) * 32
                for e16 in range(16):
                    d = dvec[e16]
                    e = (g * 16 + e16) * 32
                    tab_v[pl.ds(d, 16)] = (
                        tab_v[pl.ds(d, 16)] + rows_v[pl.ds(e, 16)])
                    tab_v[pl.ds(d + 16, 16)] = (
                        tab_v[pl.ds(d + 16, 16)] + rows_v[pl.ds(e + 16, 16)])
                return carry3

            lax.fori_loop(0, CH // 16, grp16, 0)
            return carry2

        lax.fori_loop(0, NCHUNK, body, 0)
        pltpu.sync_copy(tab_v.at[pl.ds(0, QR * 32)],
                        out_hbm.at[pl.ds((w * NPAD + qlo) * 32, QR * 32)])
        return carry

    lax.fori_loop(0, NQ, qpass, 0)


def _make_msg(din, eb):
    grid = E_PAD // eb

    def body(xs_ref, f_ref, w_ref, b_ref, r_ref):
        i = pl.program_id(0)
        xs = xs_ref[...]
        a = jnp.dot(xs, w_ref[...], preferred_element_type=jnp.float32)
        p = a * jnp.tile(f_ref[...], (1, H))
        jj = lax.broadcasted_iota(jnp.int32, (H * EF, H), 0)
        oo = lax.broadcasted_iota(jnp.int32, (H * EF, H), 1)
        sel = (jj // EF == oo).astype(jnp.float32)
        m = jnp.dot(p, sel, preferred_element_type=jnp.float32)
        m = m + jnp.dot(xs, b_ref[...], preferred_element_type=jnp.float32)
        onecol = (lax.broadcasted_iota(jnp.int32, (eb, 16), 1) == 0)
        full = jnp.concatenate([m, onecol.astype(jnp.float32)], axis=1)
        ridx = i * eb + lax.broadcasted_iota(jnp.int32, (eb, 32), 0)
        r_ref[...] = jnp.where(ridx < E, full, 0.0)

    return pl.pallas_call(
        body,
        grid=(grid,),
        in_specs=[
            pl.BlockSpec((eb, din), lambda i: (i, 0)),
            pl.BlockSpec((eb, EF), lambda i: (i, 0)),
            pl.BlockSpec((din, H * EF), lambda i: (0, 0)),
            pl.BlockSpec((din, H), lambda i: (0, 0)),
        ],
        out_specs=pl.BlockSpec((eb, 32), lambda i: (i, 0)),
        out_shape=jax.ShapeDtypeStruct((E_PAD, 32), jnp.float32),
    )


_msg1 = _make_msg(IN, 4096)


NB = 256


def _psum(p_ref):
    t = p_ref[0]
    for k in range(1, NW):
        t = t + p_ref[k]
    return t


def _norm_body_relu(p_ref, bc_ref, o_ref):
    t = _psum(p_ref)
    h = t[:, :16] / jnp.maximum(t[:, 16:17], 1.0) + bc_ref[...]
    h = jnp.maximum(h, 0.0)
    o_ref[...] = jnp.concatenate(
        [h, jnp.zeros((NB, IN - H), jnp.float32)], axis=1)


def _norm_body(p_ref, bc_ref, o_ref):
    t = _psum(p_ref)
    o_ref[...] = t[:, :16] / jnp.maximum(t[:, 16:17], 1.0) + bc_ref[...]


def _make_norm(body, d_out):
    return pl.pallas_call(
        body,
        grid=(NPAD // NB,),
        in_specs=[
            pl.BlockSpec((NW, NB, 32), lambda i: (0, i, 0)),
            pl.BlockSpec((1, H), lambda i: (0, 0)),
        ],
        out_specs=pl.BlockSpec((NB, d_out), lambda i: (i, 0)),
        out_shape=jax.ShapeDtypeStruct((NPAD, d_out), jnp.float32),
    )


_norm_relu = _make_norm(_norm_body_relu, IN)
_norm = _make_norm(_norm_body, H)


def _pad_idx(a):
    return jnp.concatenate(
        [a.astype(jnp.int32), jnp.zeros((E_PAD - E,), jnp.int32)])


def _pad_f(a):
    return jnp.concatenate(
        [a, jnp.zeros((E_PAD - E, a.shape[1]), jnp.float32)])


def kernel(edge_index1, edge_index2, in_feat, edge_feats, edge_feats1,
           W_ef1, b_ef1, W_ef2, b_ef2, b_conv1, b_conv2):
    src1, dst1 = _pad_idx(edge_index1[0]), _pad_idx(edge_index1[1])
    src2, dst2 = _pad_idx(edge_index2[0]), _pad_idx(edge_index2[1])
    f1, f2 = _pad_f(edge_feats), _pad_f(edge_feats1)
    W1d = W_ef1.reshape(IN, H * EF)
    b1d = b_ef1.reshape(IN, H)
    W2d = jnp.pad(W_ef2.reshape(H, H * EF), ((0, IN - H), (0, 0)))
    b2d = jnp.pad(b_ef2.reshape(H, H), ((0, IN - H), (0, 0)))

    xs = _gather_x(in_feat, src1)
    r1 = _msg1(xs, f1, W1d, b1d)
    p1 = _scatter_k(r1.reshape(-1), dst1).reshape(NW, NPAD, 32)
    h = _norm_relu(p1, b_conv1.reshape(1, H))
    hs = _gather_x(h, src2)
    r2 = _msg1(hs, f2, W2d, b2d)
    p2 = _scatter_k(r2.reshape(-1), dst2).reshape(NW, NPAD, 32)
    out = _norm(p2, b_conv2.reshape(1, H))
    return out[:N]


# speedup vs baseline: 1.0887x; 1.0261x over previous
import functools

import jax
import jax.numpy as jnp
from jax import lax
from jax.experimental import pallas as pl
from jax.experimental.pallas import tpu as pltpu
from jax.experimental.pallas import tpu_sc as plsc

N = 10000
E = 160000
IN = 128
H = 16
EF = 16

NC = 2
NS = 16
NW = NC * NS
E_PAD = 163840
EPW = E_PAD // NW
CH = 128
CHG = 512
NCHUNK = EPW // CH
NPAD = 10240
NQ = 4
QR = NPAD // NQ
QG = QR + 8

_mesh = plsc.VectorSubcoreMesh(core_axis_name="c", subcore_axis_name="s")


def _make_gather(d):
    @functools.partial(
        pl.kernel,
        out_type=jax.ShapeDtypeStruct((E_PAD, d), jnp.float32),
        mesh=_mesh,
        scratch_types=[
            pltpu.VMEM((CHG,), jnp.int32),
            pltpu.VMEM((CHG, d), jnp.float32),
            pltpu.SemaphoreType.DMA,
        ],
    )
    def gather_k(tab_hbm, src_hbm, out_hbm, idx_v, rows_v, sem):
        wid = lax.axis_index("s") * NC + lax.axis_index("c")
        base = wid * EPW

        def body(j, carry):
            off = base + j * CHG
            pltpu.sync_copy(src_hbm.at[pl.ds(off, CHG)], idx_v)
            pltpu.async_copy(tab_hbm.at[idx_v], rows_v, sem).wait()
            pltpu.sync_copy(rows_v, out_hbm.at[pl.ds(off, CHG)])
            return carry

        lax.fori_loop(0, EPW // CHG, body, 0)

    return gather_k


_gather_x = _make_gather(IN)


@functools.partial(
    pl.kernel,
    out_type=jax.ShapeDtypeStruct((NW * NPAD * 32,), jnp.float32),
    mesh=_mesh,
    scratch_types=[
        pltpu.VMEM((CH,), jnp.int32),
        pltpu.VMEM((CH * 32,), jnp.float32),
        pltpu.VMEM((QG * 32,), jnp.float32),
    ],
)
def _scatter_k(r_hbm, dst_hbm, out_hbm, idx_v, rows_v, tab_v):
    c = lax.axis_index("c")
    s = lax.axis_index("s")
    w = s * NC + c
    base = w * EPW

    def qpass(q, carry):
        qlo = q * QR

        def zrow(i, carry2):
            tab_v[pl.ds(i * 16, 16)] = jnp.zeros((16,), jnp.float32)
            return carry2

        lax.fori_loop(0, QG * 2, zrow, 0)

        def body(j, carry2):
            off = base + j * CH
            pltpu.sync_copy(dst_hbm.at[pl.ds(off, CH)], idx_v)
            pltpu.sync_copy(r_hbm.at[pl.ds(off * 32, CH * 32)], rows_v)

            def grp16(g, carry3):
                dvec = idx_v[pl.ds(g * 16, 16)] - qlo
                inrange = jnp.logical_and(dvec >= 0, dvec < QR)
                dvec = jnp.where(inrange, dvec, QR) * 32
                for e16 in range(16):
                    d = dvec[e16]
                    e = (g * 16 + e16) * 32
                    tab_v[pl.ds(d, 16)] = (
                        tab_v[pl.ds(d, 16)] + rows_v[pl.ds(e, 16)])
                    tab_v[pl.ds(d + 16, 16)] = (
                        tab_v[pl.ds(d + 16, 16)] + rows_v[pl.ds(e + 16, 16)])
                return carry3

            lax.fori_loop(0, CH // 16, grp16, 0)
            return carry2

        lax.fori_loop(0, NCHUNK, body, 0)
        pltpu.sync_copy(tab_v.at[pl.ds(0, QR * 32)],
                        out_hbm.at[pl.ds((w * NPAD + qlo) * 32, QR * 32)])
        return carry

    lax.fori_loop(0, NQ, qpass, 0)


def _make_msg(din, eb):
    grid = E_PAD // eb

    def body(xs_ref, f_ref, w_ref, b_ref, r_ref):
        i = pl.program_id(0)
        xs = xs_ref[...]
        a = jnp.dot(xs, w_ref[...], preferred_element_type=jnp.float32)
        p = a * jnp.tile(f_ref[...], (1, H))
        jj = lax.broadcasted_iota(jnp.int32, (H * EF, H), 0)
        oo = lax.broadcasted_iota(jnp.int32, (H * EF, H), 1)
        sel = (jj // EF == oo).astype(jnp.float32)
        m = jnp.dot(p, sel, preferred_element_type=jnp.float32)
        m = m + jnp.dot(xs, b_ref[...], preferred_element_type=jnp.float32)
        onecol = (lax.broadcasted_iota(jnp.int32, (eb, 16), 1) == 0)
        full = jnp.concatenate([m, onecol.astype(jnp.float32)], axis=1)
        ridx = i * eb + lax.broadcasted_iota(jnp.int32, (eb, 32), 0)
        r_ref[...] = jnp.where(ridx < E, full, 0.0)

    return pl.pallas_call(
        body,
        grid=(grid,),
        in_specs=[
            pl.BlockSpec((eb, din), lambda i: (i, 0)),
            pl.BlockSpec((eb, EF), lambda i: (i, 0)),
            pl.BlockSpec((din, H * EF), lambda i: (0, 0)),
            pl.BlockSpec((din, H), lambda i: (0, 0)),
        ],
        out_specs=pl.BlockSpec((eb, 32), lambda i: (i, 0)),
        out_shape=jax.ShapeDtypeStruct((E_PAD, 32), jnp.float32),
    )


_msg1 = _make_msg(IN, 4096)


NB = 256


def _psum(p_ref):
    t = p_ref[0]
    for k in range(1, NW):
        t = t + p_ref[k]
    return t


def _norm_body_relu(p_ref, bc_ref, o_ref):
    t = _psum(p_ref)
    h = t[:, :16] / jnp.maximum(t[:, 16:17], 1.0) + bc_ref[...]
    h = jnp.maximum(h, 0.0)
    o_ref[...] = jnp.concatenate(
        [h, jnp.zeros((NB, IN - H), jnp.float32)], axis=1)


def _norm_body(p_ref, bc_ref, o_ref):
    t = _psum(p_ref)
    o_ref[...] = t[:, :16] / jnp.maximum(t[:, 16:17], 1.0) + bc_ref[...]


def _make_norm(body, d_out):
    return pl.pallas_call(
        body,
        grid=(NPAD // NB,),
        in_specs=[
            pl.BlockSpec((NW, NB, 32), lambda i: (0, i, 0)),
            pl.BlockSpec((1, H), lambda i: (0, 0)),
        ],
        out_specs=pl.BlockSpec((NB, d_out), lambda i: (i, 0)),
        out_shape=jax.ShapeDtypeStruct((NPAD, d_out), jnp.float32),
    )


_norm_relu = _make_norm(_norm_body_relu, IN)
_norm = _make_norm(_norm_body, H)


def _pad_idx(a):
    return jnp.concatenate(
        [a.astype(jnp.int32), jnp.zeros((E_PAD - E,), jnp.int32)])


def _pad_f(a):
    return jnp.concatenate(
        [a, jnp.zeros((E_PAD - E, a.shape[1]), jnp.float32)])


def kernel(edge_index1, edge_index2, in_feat, edge_feats, edge_feats1,
           W_ef1, b_ef1, W_ef2, b_ef2, b_conv1, b_conv2):
    src1, dst1 = _pad_idx(edge_index1[0]), _pad_idx(edge_index1[1])
    src2, dst2 = _pad_idx(edge_index2[0]), _pad_idx(edge_index2[1])
    f1, f2 = _pad_f(edge_feats), _pad_f(edge_feats1)
    W1d = W_ef1.reshape(IN, H * EF)
    b1d = b_ef1.reshape(IN, H)
    W2d = jnp.pad(W_ef2.reshape(H, H * EF), ((0, IN - H), (0, 0)))
    b2d = jnp.pad(b_ef2.reshape(H, H), ((0, IN - H), (0, 0)))

    xs = _gather_x(in_feat, src1)
    r1 = _msg1(xs, f1, W1d, b1d)
    p1 = _scatter_k(r1.reshape(-1), dst1).reshape(NW, NPAD, 32)
    h = _norm_relu(p1, b_conv1.reshape(1, H))
    hs = _gather_x(h, src2)
    r2 = _msg1(hs, f2, W2d, b2d)
    p2 = _scatter_k(r2.reshape(-1), dst2).reshape(NW, NPAD, 32)
    out = _norm(p2, b_conv2.reshape(1, H))
    return out[:N]


# speedup vs baseline: 1.4216x; 1.3057x over previous
import functools

import jax
import jax.numpy as jnp
from jax import lax
from jax.experimental import pallas as pl
from jax.experimental.pallas import tpu as pltpu
from jax.experimental.pallas import tpu_sc as plsc

N = 10000
E = 160000
IN = 128
H = 16
EF = 16

NC = 2
NS = 16
NW = NC * NS
E_PAD = 163840
EPW = E_PAD // NW
CH = 128
CHG = 512
NCHUNK = EPW // CH
NPAD = 10240
NQ = 2
QR = NPAD // NQ
QG = QR + 8
DGR = NPAD // 16

_mesh = plsc.VectorSubcoreMesh(core_axis_name="c", subcore_axis_name="s")


def _make_gather(d):
    @functools.partial(
        pl.kernel,
        out_type=jax.ShapeDtypeStruct((E_PAD, d), jnp.float32),
        mesh=_mesh,
        scratch_types=[
            pltpu.VMEM((CHG,), jnp.int32),
            pltpu.VMEM((CHG, d), jnp.float32),
            pltpu.SemaphoreType.DMA,
        ],
    )
    def gather_k(tab_hbm, src_hbm, out_hbm, idx_v, rows_v, sem):
        wid = lax.axis_index("s") * NC + lax.axis_index("c")
        base = wid * EPW

        def body(j, carry):
            off = base + j * CHG
            pltpu.sync_copy(src_hbm.at[pl.ds(off, CHG)], idx_v)
            pltpu.async_copy(tab_hbm.at[idx_v], rows_v, sem).wait()
            pltpu.sync_copy(rows_v, out_hbm.at[pl.ds(off, CHG)])
            return carry

        lax.fori_loop(0, EPW // CHG, body, 0)

    return gather_k


_gather_x = _make_gather(IN)


@functools.partial(
    pl.kernel,
    out_type=[jax.ShapeDtypeStruct((NW * NPAD * 16,), jnp.float32),
              jax.ShapeDtypeStruct((NW * NPAD,), jnp.float32)],
    mesh=_mesh,
    scratch_types=[
        pltpu.VMEM((CH,), jnp.int32),
        pltpu.VMEM((CH * 16,), jnp.float32),
        pltpu.VMEM((QG * 16,), jnp.float32),
        pltpu.VMEM((DGR * 16,), jnp.float32),
    ],
)
def _scatter_k(r_hbm, dst_hbm, outm_hbm, outd_hbm,
               idx_v, rows_v, tab_v, deg_v):
    c = lax.axis_index("c")
    s = lax.axis_index("s")
    w = s * NC + c
    base = w * EPW
    iota16 = lax.iota(jnp.int32, 16)

    for q in range(NQ):
        qlo = q * QR

        def zrow(i, carry2):
            tab_v[pl.ds(i * 16, 16)] = jnp.zeros((16,), jnp.float32)
            return carry2

        lax.fori_loop(0, QG, zrow, 0)
        if q == 0:
            def zdeg(i, carry2):
                deg_v[pl.ds(i * 16, 16)] = jnp.zeros((16,), jnp.float32)
                return carry2

            lax.fori_loop(0, DGR, zdeg, 0)

        def body(j, carry2, q=q, qlo=qlo):
            off = base + j * CH
            pltpu.sync_copy(dst_hbm.at[pl.ds(off, CH)], idx_v)
            pltpu.sync_copy(r_hbm.at[pl.ds(off * 16, CH * 16)], rows_v)

            def grp16(g, carry3):
                dvec = idx_v[pl.ds(g * 16, 16)]
                dm = dvec - qlo
                inrange = jnp.logical_and(dm >= 0, dm < QR)
                dmi = jnp.where(inrange, dm, QR) * 16
                for e16 in range(16):
                    d = dmi[e16]
                    e = (g * 16 + e16) * 16
                    tab_v[pl.ds(d, 16)] = (
                        tab_v[pl.ds(d, 16)] + rows_v[pl.ds(e, 16)])
                if q == 0:
                    rvec = (dvec >> 4) * 16
                    lvec = dvec & 15
                    for e16 in range(16):
                        dr = rvec[e16]
                        oh = jnp.where(iota16 == lvec[e16], 1.0, 0.0)
                        deg_v[pl.ds(dr, 16)] = deg_v[pl.ds(dr, 16)] + oh
                return carry3

            lax.fori_loop(0, CH // 16, grp16, 0)
            return carry2

        lax.fori_loop(0, NCHUNK, body, 0)
        pltpu.sync_copy(tab_v.at[pl.ds(0, QR * 16)],
                        outm_hbm.at[pl.ds((w * NPAD + qlo) * 16, QR * 16)])

    pltpu.sync_copy(deg_v, outd_hbm.at[pl.ds(w * NPAD, NPAD)])


def _make_msg(din, eb):
    grid = E_PAD // eb

    def body(xs_ref, f_ref, w_ref, b_ref, r_ref):
        i = pl.program_id(0)
        xs = xs_ref[...]
        a = jnp.dot(xs, w_ref[...], preferred_element_type=jnp.float32)
        p = a * jnp.tile(f_ref[...], (1, H))
        jj = lax.broadcasted_iota(jnp.int32, (H * EF, H), 0)
        oo = lax.broadcasted_iota(jnp.int32, (H * EF, H), 1)
        sel = (jj // EF == oo).astype(jnp.float32)
        m = jnp.dot(p, sel, preferred_element_type=jnp.float32)
        m = m + jnp.dot(xs, b_ref[...], preferred_element_type=jnp.float32)
        ridx = i * eb + lax.broadcasted_iota(jnp.int32, (eb, H), 0)
        r_ref[...] = jnp.where(ridx < E, m, 0.0)

    return pl.pallas_call(
        body,
        grid=(grid,),
        in_specs=[
            pl.BlockSpec((eb, din), lambda i: (i, 0)),
            pl.BlockSpec((eb, EF), lambda i: (i, 0)),
            pl.BlockSpec((din, H * EF), lambda i: (0, 0)),
            pl.BlockSpec((din, H), lambda i: (0, 0)),
        ],
        out_specs=pl.BlockSpec((eb, H), lambda i: (i, 0)),
        out_shape=jax.ShapeDtypeStruct((E_PAD, H), jnp.float32),
    )


_msg1 = _make_msg(IN, 4096)


NB = 512


def _psum(p_ref):
    t = p_ref[0]
    for k in range(1, NW):
        t = t + p_ref[k]
    return t


def _degsum_body(pd_ref, o_ref):
    o_ref[...] = _psum(pd_ref)


_degsum = pl.pallas_call(
    _degsum_body,
    out_shape=jax.ShapeDtypeStruct((DGR, 16), jnp.float32))


def _norm_body_relu(pm_ref, dg_ref, bc_ref, o_ref):
    t = _psum(pm_ref)
    h = t / jnp.maximum(dg_ref[...], 1.0) + bc_ref[...]
    h = jnp.maximum(h, 0.0)
    o_ref[...] = jnp.concatenate(
        [h, jnp.zeros((NB, IN - H), jnp.float32)], axis=1)


def _norm_body(pm_ref, dg_ref, bc_ref, o_ref):
    t = _psum(pm_ref)
    o_ref[...] = t / jnp.maximum(dg_ref[...], 1.0) + bc_ref[...]


def _make_norm(body, d_out):
    return pl.pallas_call(
        body,
        grid=(NPAD // NB,),
        in_specs=[
            pl.BlockSpec((NW, NB, H), lambda i: (0, i, 0)),
            pl.BlockSpec((NB, H), lambda i: (i, 0)),
            pl.BlockSpec((1, H), lambda i: (0, 0)),
        ],
        out_specs=pl.BlockSpec((NB, d_out), lambda i: (i, 0)),
        out_shape=jax.ShapeDtypeStruct((NPAD, d_out), jnp.float32),
    )


_norm_relu = _make_norm(_norm_body_relu, IN)
_norm = _make_norm(_norm_body, H)


def _pad_idx(a, fill=0):
    return jnp.concatenate(
        [a.astype(jnp.int32),
         jnp.full((E_PAD - E,), fill, jnp.int32)])


def _pad_f(a):
    return jnp.concatenate(
        [a, jnp.zeros((E_PAD - E, a.shape[1]), jnp.float32)])


def kernel(edge_index1, edge_index2, in_feat, edge_feats, edge_feats1,
           W_ef1, b_ef1, W_ef2, b_ef2, b_conv1, b_conv2):
    src1, dst1 = _pad_idx(edge_index1[0]), _pad_idx(edge_index1[1], NPAD - 1)
    src2, dst2 = _pad_idx(edge_index2[0]), _pad_idx(edge_index2[1], NPAD - 1)
    f1, f2 = _pad_f(edge_feats), _pad_f(edge_feats1)
    W1d = W_ef1.reshape(IN, H * EF)
    b1d = b_ef1.reshape(IN, H)
    W2d = jnp.pad(W_ef2.reshape(H, H * EF), ((0, IN - H), (0, 0)))
    b2d = jnp.pad(b_ef2.reshape(H, H), ((0, IN - H), (0, 0)))

    def _agg(r, dst):
        pm, pd = _scatter_k(r.reshape(-1), dst)
        degsum = _degsum(pd.reshape(NW, DGR, 16))
        degf = jnp.broadcast_to(degsum.reshape(NPAD, 1), (NPAD, H))
        return pm.reshape(NW, NPAD, H), degf

    xs = _gather_x(in_feat, src1)
    r1 = _msg1(xs, f1, W1d, b1d)
    pm1, degf1 = _agg(r1, dst1)
    h = _norm_relu(pm1, degf1, b_conv1.reshape(1, H))
    hs = _gather_x(h, src2)
    r2 = _msg1(hs, f2, W2d, b2d)
    pm2, degf2 = _agg(r2, dst2)
    out = _norm(pm2, degf2, b_conv2.reshape(1, H))
    return out[:N]


# speedup vs baseline: 1.4597x; 1.0268x over previous
import functools

import jax
import jax.numpy as jnp
from jax import lax
from jax.experimental import pallas as pl
from jax.experimental.pallas import tpu as pltpu
from jax.experimental.pallas import tpu_sc as plsc

N = 10000
E = 160000
IN = 128
H = 16
EF = 16

NC = 2
NS = 16
NW = NC * NS
E_PAD = 163840
EPW = E_PAD // NW
CH = 128
CHG = 256
NPAIR = EPW // (2 * CHG)
NCHUNK = EPW // CH
NPAD = 10240
NQ = 2
QR = NPAD // NQ
QG = QR + 8
DGR = NPAD // 16

_mesh = plsc.VectorSubcoreMesh(core_axis_name="c", subcore_axis_name="s")


def _make_gather(d):
    @functools.partial(
        pl.kernel,
        out_type=jax.ShapeDtypeStruct((E_PAD, d), jnp.float32),
        mesh=_mesh,
        scratch_types=[
            pltpu.VMEM((CHG,), jnp.int32),
            pltpu.VMEM((CHG,), jnp.int32),
            pltpu.VMEM((CHG, d), jnp.float32),
            pltpu.VMEM((CHG, d), jnp.float32),
            pltpu.SemaphoreType.DMA,
            pltpu.SemaphoreType.DMA,
            pltpu.SemaphoreType.DMA,
            pltpu.SemaphoreType.DMA,
        ],
    )
    def gather_k(tab_hbm, src_hbm, out_hbm,
                 idxa, idxb, rowsa, rowsb, gsa, gsb, wsa, wsb):
        wid = lax.axis_index("s") * NC + lax.axis_index("c")
        base = wid * EPW

        pltpu.sync_copy(src_hbm.at[pl.ds(base, CHG)], idxa)
        pltpu.async_copy(tab_hbm.at[idxa], rowsa, gsa)

        def pair(p, carry):
            ca = base + (2 * p) * CHG
            cb = ca + CHG
            pltpu.sync_copy(src_hbm.at[pl.ds(cb, CHG)], idxb)

            @pl.when(p > 0)
            def _():
                pltpu.make_async_copy(
                    rowsb, out_hbm.at[pl.ds(ca - CHG, CHG)], wsb).wait()

            pltpu.async_copy(tab_hbm.at[idxb], rowsb, gsb)
            pltpu.make_async_copy(tab_hbm.at[idxa], rowsa, gsa).wait()
            pltpu.async_copy(rowsa, out_hbm.at[pl.ds(ca, CHG)], wsa)

            @pl.when(p + 1 < NPAIR)
            def _():
                pltpu.sync_copy(src_hbm.at[pl.ds(ca + 2 * CHG, CHG)], idxa)
                pltpu.make_async_copy(
                    rowsa, out_hbm.at[pl.ds(ca, CHG)], wsa).wait()
                pltpu.async_copy(tab_hbm.at[idxa], rowsa, gsa)

            pltpu.make_async_copy(tab_hbm.at[idxb], rowsb, gsb).wait()
            pltpu.async_copy(rowsb, out_hbm.at[pl.ds(cb, CHG)], wsb)
            return carry

        lax.fori_loop(0, NPAIR, pair, 0)
        last = base + (2 * NPAIR - 2) * CHG
        pltpu.make_async_copy(
            rowsa, out_hbm.at[pl.ds(last, CHG)], wsa).wait()
        pltpu.make_async_copy(
            rowsb, out_hbm.at[pl.ds(last + CHG, CHG)], wsb).wait()

    return gather_k


_gather_x = _make_gather(IN)


@functools.partial(
    pl.kernel,
    out_type=[jax.ShapeDtypeStruct((NW * NPAD * 16,), jnp.float32),
              jax.ShapeDtypeStruct((NW * NPAD,), jnp.float32)],
    mesh=_mesh,
    scratch_types=[
        pltpu.VMEM((CH,), jnp.int32),
        pltpu.VMEM((CH * 16,), jnp.float32),
        pltpu.VMEM((QG * 16,), jnp.float32),
        pltpu.VMEM((DGR * 16,), jnp.float32),
    ],
)
def _scatter_k(r_hbm, dst_hbm, outm_hbm, outd_hbm,
               idx_v, rows_v, tab_v, deg_v):
    c = lax.axis_index("c")
    s = lax.axis_index("s")
    w = s * NC + c
    base = w * EPW
    iota16 = lax.iota(jnp.int32, 16)

    for q in range(NQ):
        qlo = q * QR

        def zrow(i, carry2):
            tab_v[pl.ds(i * 16, 16)] = jnp.zeros((16,), jnp.float32)
            return carry2

        lax.fori_loop(0, QG, zrow, 0)
        if q == 0:
            def zdeg(i, carry2):
                deg_v[pl.ds(i * 16, 16)] = jnp.zeros((16,), jnp.float32)
                return carry2

            lax.fori_loop(0, DGR, zdeg, 0)

        def body(j, carry2, q=q, qlo=qlo):
            off = base + j * CH
            pltpu.sync_copy(dst_hbm.at[pl.ds(off, CH)], idx_v)
            pltpu.sync_copy(r_hbm.at[pl.ds(off * 16, CH * 16)], rows_v)

            def grp16(g, carry3):
                dvec = idx_v[pl.ds(g * 16, 16)]
                dm = dvec - qlo
                inrange = jnp.logical_and(dm >= 0, dm < QR)
                dmi = jnp.where(inrange, dm, QR) * 16
                for e16 in range(16):
                    d = dmi[e16]
                    e = (g * 16 + e16) * 16
                    tab_v[pl.ds(d, 16)] = (
                        tab_v[pl.ds(d, 16)] + rows_v[pl.ds(e, 16)])
                if q == 0:
                    rvec = (dvec >> 4) * 16
                    lvec = dvec & 15
                    for e16 in range(16):
                        dr = rvec[e16]
                        oh = jnp.where(iota16 == lvec[e16], 1.0, 0.0)
                        deg_v[pl.ds(dr, 16)] = deg_v[pl.ds(dr, 16)] + oh
                return carry3

            lax.fori_loop(0, CH // 16, grp16, 0)
            return carry2

        lax.fori_loop(0, NCHUNK, body, 0)
        pltpu.sync_copy(tab_v.at[pl.ds(0, QR * 16)],
                        outm_hbm.at[pl.ds((w * NPAD + qlo) * 16, QR * 16)])

    pltpu.sync_copy(deg_v, outd_hbm.at[pl.ds(w * NPAD, NPAD)])


def _make_msg(din, eb):
    grid = E_PAD // eb

    def body(xs_ref, f_ref, w_ref, b_ref, r_ref):
        i = pl.program_id(0)
        xs = xs_ref[...]
        a = jnp.dot(xs, w_ref[...], preferred_element_type=jnp.float32)
        p = a * jnp.tile(f_ref[...], (1, H))
        jj = lax.broadcasted_iota(jnp.int32, (H * EF, H), 0)
        oo = lax.broadcasted_iota(jnp.int32, (H * EF, H), 1)
        sel = (jj // EF == oo).astype(jnp.float32)
        m = jnp.dot(p, sel, preferred_element_type=jnp.float32)
        m = m + jnp.dot(xs, b_ref[...], preferred_element_type=jnp.float32)
        ridx = i * eb + lax.broadcasted_iota(jnp.int32, (eb, H), 0)
        r_ref[...] = jnp.where(ridx < E, m, 0.0)

    return pl.pallas_call(
        body,
        grid=(grid,),
        in_specs=[
            pl.BlockSpec((eb, din), lambda i: (i, 0)),
            pl.BlockSpec((eb, EF), lambda i: (i, 0)),
            pl.BlockSpec((din, H * EF), lambda i: (0, 0)),
            pl.BlockSpec((din, H), lambda i: (0, 0)),
        ],
        out_specs=pl.BlockSpec((eb, H), lambda i: (i, 0)),
        out_shape=jax.ShapeDtypeStruct((E_PAD, H), jnp.float32),
    )


_msg1 = _make_msg(IN, 4096)


NB = 512


def _psum(p_ref):
    t = p_ref[0]
    for k in range(1, NW):
        t = t + p_ref[k]
    return t


def _degsum_body(pd_ref, o_ref):
    o_ref[...] = _psum(pd_ref)


_degsum = pl.pallas_call(
    _degsum_body,
    out_shape=jax.ShapeDtypeStruct((DGR, 16), jnp.float32))


def _norm_body_relu(pm_ref, dg_ref, bc_ref, o_ref):
    t = _psum(pm_ref)
    h = t / jnp.maximum(dg_ref[...], 1.0) + bc_ref[...]
    h = jnp.maximum(h, 0.0)
    o_ref[...] = jnp.concatenate(
        [h, jnp.zeros((NB, IN - H), jnp.float32)], axis=1)


def _norm_body(pm_ref, dg_ref, bc_ref, o_ref):
    t = _psum(pm_ref)
    o_ref[...] = t / jnp.maximum(dg_ref[...], 1.0) + bc_ref[...]


def _make_norm(body, d_out):
    return pl.pallas_call(
        body,
        grid=(NPAD // NB,),
        in_specs=[
            pl.BlockSpec((NW, NB, H), lambda i: (0, i, 0)),
            pl.BlockSpec((NB, H), lambda i: (i, 0)),
            pl.BlockSpec((1, H), lambda i: (0, 0)),
        ],
        out_specs=pl.BlockSpec((NB, d_out), lambda i: (i, 0)),
        out_shape=jax.ShapeDtypeStruct((NPAD, d_out), jnp.float32),
    )


_norm_relu = _make_norm(_norm_body_relu, IN)
_norm = _make_norm(_norm_body, H)


def _pad_idx(a, fill=0):
    return jnp.concatenate(
        [a.astype(jnp.int32),
         jnp.full((E_PAD - E,), fill, jnp.int32)])


def _pad_f(a):
    return jnp.concatenate(
        [a, jnp.zeros((E_PAD - E, a.shape[1]), jnp.float32)])


def kernel(edge_index1, edge_index2, in_feat, edge_feats, edge_feats1,
           W_ef1, b_ef1, W_ef2, b_ef2, b_conv1, b_conv2):
    src1, dst1 = _pad_idx(edge_index1[0]), _pad_idx(edge_index1[1], NPAD - 1)
    src2, dst2 = _pad_idx(edge_index2[0]), _pad_idx(edge_index2[1], NPAD - 1)
    f1, f2 = _pad_f(edge_feats), _pad_f(edge_feats1)
    W1d = W_ef1.reshape(IN, H * EF)
    b1d = b_ef1.reshape(IN, H)
    W2d = jnp.pad(W_ef2.reshape(H, H * EF), ((0, IN - H), (0, 0)))
    b2d = jnp.pad(b_ef2.reshape(H, H), ((0, IN - H), (0, 0)))

    def _agg(r, dst):
        pm, pd = _scatter_k(r.reshape(-1), dst)
        degsum = _degsum(pd.reshape(NW, DGR, 16))
        degf = jnp.broadcast_to(degsum.reshape(NPAD, 1), (NPAD, H))
        return pm.reshape(NW, NPAD, H), degf

    xs = _gather_x(in_feat, src1)
    r1 = _msg1(xs, f1, W1d, b1d)
    pm1, degf1 = _agg(r1, dst1)
    h = _norm_relu(pm1, degf1, b_conv1.reshape(1, H))
    hs = _gather_x(h, src2)
    r2 = _msg1(hs, f2, W2d, b2d)
    pm2, degf2 = _agg(r2, dst2)
    out = _norm(pm2, degf2, b_conv2.reshape(1, H))
    return out[:N]
